# baseline (speedup 1.0000x reference)
import functools

import jax
import jax.numpy as jnp
from jax import lax
from jax.experimental import pallas as pl
from jax.experimental.pallas import tpu as pltpu
from jax.experimental.pallas import tpu_sc as plsc

_VOCAB = 1000000
_EMBED = 64
_BATCH = 16384
_POS = 10
_NEG = 50
_NLAB = 64

_NC, _NS = 2, 16
_NW = _NC * _NS
_BPW = _BATCH // _NW
_C = 4
_NCHUNK = _BPW // _C
_RPC = _C * _NLAB
_NSTREAM = _RPC // 128
_LROW = _RPC // 128

_mesh = plsc.VectorSubcoreMesh(core_axis_name="c", subcore_axis_name="s")


@functools.partial(
    pl.kernel,
    out_type=jax.ShapeDtypeStruct((_BATCH * _NLAB // 128, 128), jnp.float32),
    mesh=_mesh,
    compiler_params=pltpu.CompilerParams(
        needs_layout_passes=False, use_tc_tiling_on_sc=True),
    scratch_types=[
        pltpu.VMEM((2, _LROW, 128), jnp.int32),
        pltpu.VMEM((2, _LROW, 128), jnp.int32),
        pltpu.VMEM((2, _LROW, 128), jnp.int32),
        pltpu.VMEM((2, _RPC, 128), jnp.float32),
        pltpu.VMEM((2, _C * _EMBED // 128, 128), jnp.float32),
        pltpu.VMEM((2, _LROW, 128), jnp.float32),
        pltpu.SemaphoreType.DMA((2,)),
        pltpu.SemaphoreType.DMA((2,)),
        pltpu.SemaphoreType.DMA((2,)),
    ],
)
def _sc_scores(labels_hbm, xg_hbm, outemb_hbm, out_hbm,
               lab_v, pidx_v, side_v, rows_v, x_v, sc_v,
               sem_lab, sem_gat, sem_out):
  wid = lax.axis_index("s") * _NC + lax.axis_index("c")
  b0 = wid * _BPW
  r0 = wid * (_BPW * _NLAB // 128)
  xr0 = wid * (_BPW * _EMBED // 128)
  _XR = _C * _EMBED // 128

  def issue_idx(c, p):
    pltpu.async_copy(labels_hbm.at[pl.ds(r0 + c * _LROW, _LROW)],
                     lab_v.at[p], sem_lab.at[p])
    pltpu.async_copy(xg_hbm.at[pl.ds(xr0 + c * _XR, _XR)],
                     x_v.at[p], sem_lab.at[p])

  def wait_idx(p):
    pltpu.make_async_copy(labels_hbm.at[pl.ds(0, _LROW)],
                          lab_v.at[p], sem_lab.at[p]).wait()
    pltpu.make_async_copy(xg_hbm.at[pl.ds(0, _XR)], x_v.at[p],
                          sem_lab.at[p]).wait()

  def derive_idx(p):
    for k in range(_LROW):
      for u in range(8):
        raw = lab_v[p, k, pl.ds(u * 16, 16)]
        pidx_v[p, k, pl.ds(u * 16, 16)] = raw >> 1
        side_v[p, k, pl.ds(u * 16, 16)] = raw & 1

  def issue_gather(p):
    for k in range(_NSTREAM):
      pltpu.async_copy(outemb_hbm.at[pidx_v.at[p, k]],
                       rows_v.at[p, pl.ds(k * 128, 128)], sem_gat.at[p])

  def wait_gather(p):
    for k in range(_NSTREAM):
      pltpu.make_async_copy(outemb_hbm.at[pidx_v.at[p, k]],
                            rows_v.at[p, pl.ds(k * 128, 128)],
                            sem_gat.at[p]).wait()

  def issue_out(c, p):
    pltpu.async_copy(sc_v.at[p], out_hbm.at[pl.ds(r0 + c * _LROW, _LROW)],
                     sem_out.at[p])

  def wait_out(p):
    pltpu.make_async_copy(sc_v.at[p], out_hbm.at[pl.ds(0, _LROW)],
                          sem_out.at[p]).wait()

  iota = lax.iota(jnp.int32, 16)

  def compute(p):
    def per_b(b, carry):
      xv = [x_v[p, b >> 1, pl.ds((b & 1) * _EMBED + cc * 16, 16)]
            for cc in range(_EMBED // 16)]
      for g in range(_NLAB // 16):
        acc_lo = jnp.zeros((16,), jnp.float32)
        acc_hi = jnp.zeros((16,), jnp.float32)
        for j in range(16):
          r = b * _NLAB + g * 16 + j
          dot_lo = rows_v[p, r, pl.ds(0, 16)] * xv[0]
          dot_hi = rows_v[p, r, pl.ds(_EMBED, 16)] * xv[0]
          for cc in range(1, _EMBED // 16):
            dot_lo = dot_lo + rows_v[p, r, pl.ds(cc * 16, 16)] * xv[cc]
            dot_hi = dot_hi + rows_v[p, r, pl.ds(_EMBED + cc * 16, 16)] * xv[cc]
          s_lo = jnp.sum(dot_lo)
          s_hi = jnp.sum(dot_hi)
          acc_lo = jnp.where(iota == j, s_lo, acc_lo)
          acc_hi = jnp.where(iota == j, s_hi, acc_hi)
        slot = b * _NLAB + g * 16
        svec = side_v[p, slot >> 7, pl.ds(slot & 127, 16)]
        sc_v[p, slot >> 7, pl.ds(slot & 127, 16)] = jnp.where(
            svec == 1, acc_hi, acc_lo)
      return carry

    lax.fori_loop(0, _C, per_b, 0)

  issue_idx(0, 0)
  wait_idx(0)
  derive_idx(0)
  issue_gather(0)
  issue_idx(1, 1)

  def step(i, carry):
    for p in (0, 1):
      c = i * 2 + p

      @pl.when(c + 1 < _NCHUNK)
      def _():
        wait_idx(1 - p)
        derive_idx(1 - p)
        issue_gather(1 - p)

      wait_gather(p)

      @pl.when(c + 2 < _NCHUNK)
      def _():
        issue_idx(c + 2, p)

      @pl.when(c >= 2)
      def _():
        wait_out(p)

      compute(p)
      issue_out(c, p)
    return carry

  lax.fori_loop(0, _NCHUNK // 2, step, 0)
  wait_out(0)
  wait_out(1)


_TC_ROWS = 4096


def _tc_loss_body(s_ref, o_ref):
  s = s_ref[...]
  col = lax.broadcasted_iota(jnp.int32, (_TC_ROWS, _NLAB), 1)
  z = jnp.where(col < _POS, s, -s)
  ls = jnp.minimum(z, 0.0) - jnp.log1p(jnp.exp(-jnp.abs(z)))
  contrib = jnp.where(col < _POS + _NEG, ls, 0.0)
  o_ref[...] = -jnp.sum(contrib, axis=1).reshape(_TC_ROWS // 512, 512)


_tc_loss = pl.pallas_call(
    _tc_loss_body,
    grid=(_BATCH // _TC_ROWS,),
    in_specs=[pl.BlockSpec((_TC_ROWS, _NLAB), lambda i: (i, 0))],
    out_specs=pl.BlockSpec((_TC_ROWS // 512, 512), lambda i: (i, 0)),
    out_shape=jax.ShapeDtypeStruct((_BATCH // 512, 512), jnp.float32),
)


@jax.jit
def _impl(input_labels, pos_labels, neg_labels, in_embed, out_embed):
  pad = jnp.broadcast_to(pos_labels[:, :1], (_BATCH, _NLAB - _POS - _NEG))
  labels = jnp.concatenate(
      [pos_labels, neg_labels, pad], axis=1).astype(jnp.int32)
  labels2d = labels.reshape(_BATCH * _NLAB // 128, 128)
  xg = jnp.take(in_embed, input_labels, axis=0)
  xg2d = xg.reshape(_BATCH * _EMBED // 128, 128)
  out2 = out_embed.reshape(_VOCAB // 2, 2 * _EMBED)
  scores = _sc_scores(labels2d, xg2d, out2)
  return _tc_loss(scores.reshape(_BATCH, _NLAB)).reshape(_BATCH)


def kernel(input_labels, pos_labels, neg_labels, in_embed, out_embed):
  return _impl(input_labels, pos_labels, neg_labels, in_embed, out_embed)


# speedup vs baseline: 1.1079x; 1.1079x over previous
import functools

import jax
import jax.numpy as jnp
from jax import lax
from jax.experimental import pallas as pl
from jax.experimental.pallas import tpu as pltpu
from jax.experimental.pallas import tpu_sc as plsc

_VOCAB = 1000000
_EMBED = 64
_BATCH = 16384
_POS = 10
_NEG = 50
_NLAB = 64

_NC, _NS = 2, 16
_NW = _NC * _NS
_BPW = _BATCH // _NW
_C = 8
_NCHUNK = _BPW // _C
_RPC = _C * _NLAB
_NSTREAM = _RPC // 128

_mesh = plsc.VectorSubcoreMesh(core_axis_name="c", subcore_axis_name="s")


@functools.partial(
    pl.kernel,
    out_type=jax.ShapeDtypeStruct((_BATCH, _NLAB), jnp.float32),
    mesh=_mesh,
    compiler_params=pltpu.CompilerParams(
        needs_layout_passes=False, use_tc_tiling_on_sc=False),
    scratch_types=[
        pltpu.VMEM((2, _NSTREAM, 128), jnp.int32),
        pltpu.VMEM((2, _RPC, _EMBED), jnp.float32),
        pltpu.VMEM((2, _C, _EMBED), jnp.float32),
        pltpu.VMEM((2, _C, _NLAB), jnp.float32),
        pltpu.SemaphoreType.DMA((2,)),
        pltpu.SemaphoreType.DMA((2,)),
        pltpu.SemaphoreType.DMA((2,)),
    ],
)
def _sc_scores(labels_hbm, outemb_hbm, xg_hbm, out_hbm,
               lab_v, rows_v, x_v, sc_v,
               sem_lab, sem_gat, sem_out):
  wid = lax.axis_index("s") * _NC + lax.axis_index("c")
  b0 = wid * _BPW
  r0 = wid * (_BPW * _NLAB // 128)

  def issue_idx(c, p):
    pltpu.async_copy(labels_hbm.at[pl.ds(r0 + c * _NSTREAM, _NSTREAM)],
                     lab_v.at[p], sem_lab.at[p])
    pltpu.async_copy(xg_hbm.at[pl.ds(b0 + c * _C, _C)],
                     x_v.at[p], sem_lab.at[p])

  def wait_idx(p):
    pltpu.make_async_copy(labels_hbm.at[pl.ds(0, _NSTREAM)],
                          lab_v.at[p], sem_lab.at[p]).wait()
    pltpu.make_async_copy(xg_hbm.at[pl.ds(0, _C)], x_v.at[p],
                          sem_lab.at[p]).wait()

  def issue_gather(p):
    for k in range(_NSTREAM):
      pltpu.async_copy(
          outemb_hbm.at[lab_v.at[p, k]],
          rows_v.at[p, pl.ds(k * 128, 128)], sem_gat.at[p])

  def wait_gather(p):
    for k in range(_NSTREAM):
      pltpu.make_async_copy(
          outemb_hbm.at[lab_v.at[p, k]],
          rows_v.at[p, pl.ds(k * 128, 128)],
          sem_gat.at[p]).wait()

  def issue_out(c, p):
    pltpu.async_copy(sc_v.at[p], out_hbm.at[pl.ds(b0 + c * _C, _C)],
                     sem_out.at[p])

  def wait_out(p):
    pltpu.make_async_copy(sc_v.at[p], out_hbm.at[pl.ds(0, _C)],
                          sem_out.at[p]).wait()

  iota = lax.iota(jnp.int32, 16)

  def compute(p):
    def per_b(b, carry):
      xv = [x_v[p, b, pl.ds(cc * 16, 16)] for cc in range(_EMBED // 16)]
      for g in range(_NLAB // 16):
        acc = jnp.zeros((16,), jnp.float32)
        for j in range(16):
          r = b * _NLAB + g * 16 + j
          prod = rows_v[p, r, pl.ds(0, 16)] * xv[0]
          for cc in range(1, _EMBED // 16):
            prod = prod + rows_v[p, r, pl.ds(cc * 16, 16)] * xv[cc]
          s = jnp.sum(prod)
          acc = jnp.where(iota == j, s, acc)
        sc_v[p, b, pl.ds(g * 16, 16)] = acc
      return carry

    lax.fori_loop(0, _C, per_b, 0)

  issue_idx(0, 0)
  wait_idx(0)
  issue_gather(0)
  issue_idx(1, 1)

  def step(i, carry):
    for p in (0, 1):
      c = i * 2 + p

      @pl.when(c + 1 < _NCHUNK)
      def _():
        wait_idx(1 - p)
        issue_gather(1 - p)

      wait_gather(p)

      @pl.when(c + 2 < _NCHUNK)
      def _():
        issue_idx(c + 2, p)

      @pl.when(c >= 2)
      def _():
        wait_out(p)

      compute(p)
      issue_out(c, p)
    return carry

  lax.fori_loop(0, _NCHUNK // 2, step, 0)
  wait_out(0)
  wait_out(1)


_TC_ROWS = 4096


def _tc_loss_body(s_ref, o_ref):
  s = s_ref[...]
  col = lax.broadcasted_iota(jnp.int32, (_TC_ROWS, _NLAB), 1)
  z = jnp.where(col < _POS, s, -s)
  ls = jnp.minimum(z, 0.0) - jnp.log1p(jnp.exp(-jnp.abs(z)))
  contrib = jnp.where(col < _POS + _NEG, ls, 0.0)
  o_ref[...] = -jnp.sum(contrib, axis=1).reshape(_TC_ROWS // 512, 512)


_tc_loss = pl.pallas_call(
    _tc_loss_body,
    grid=(_BATCH // _TC_ROWS,),
    in_specs=[pl.BlockSpec((_TC_ROWS, _NLAB), lambda i: (i, 0))],
    out_specs=pl.BlockSpec((_TC_ROWS // 512, 512), lambda i: (i, 0)),
    out_shape=jax.ShapeDtypeStruct((_BATCH // 512, 512), jnp.float32),
)


@jax.jit
def _impl(input_labels, pos_labels, neg_labels, in_embed, out_embed):
  pad = jnp.broadcast_to(pos_labels[:, :1], (_BATCH, _NLAB - _POS - _NEG))
  labels = jnp.concatenate(
      [pos_labels, neg_labels, pad], axis=1).astype(jnp.int32)
  labels2d = labels.reshape(_BATCH * _NLAB // 128, 128)
  xg = jnp.take(in_embed, input_labels, axis=0)
  scores = _sc_scores(labels2d, out_embed, xg)
  return _tc_loss(scores).reshape(_BATCH)


def kernel(input_labels, pos_labels, neg_labels, in_embed, out_embed):
  return _impl(input_labels, pos_labels, neg_labels, in_embed, out_embed)
